# Initial kernel scaffold; baseline (speedup 1.0000x reference)
import jax
import jax.numpy as jnp
from jax.experimental import pallas as pl


def kernel(class_emb, rel_emb, nf1, nf2, nf3, nf4, disjoint, neg, top):
    raise NotImplementedError("write your pallas kernel here")


# speedup vs baseline: 1.1606x; 1.1606x over previous
import functools

import jax
import jax.numpy as jnp
from jax import lax
from jax.experimental import pallas as pl
from jax.experimental.pallas import tpu as pltpu
from jax.experimental.pallas import tpu_sc as plsc

_DIM = 128
_B = 512
_MARGIN = 0.1
_MARGIN1 = 0.05
_INF = 4.0
_NSLOT = 16
_NROWS = _NSLOT * _B
_NW = 32
_RPW = _NROWS // _NW
_CHUNK = 128

@functools.cache
def _get_sc_gather():
    mesh = plsc.VectorSubcoreMesh(core_axis_name="c", subcore_axis_name="s")

    @functools.partial(
        pl.kernel,
        mesh=mesh,
        out_type=jax.ShapeDtypeStruct((_NROWS, 2 * _DIM), jnp.float32),
        scratch_types=[
            pltpu.VMEM((_CHUNK,), jnp.int32),
            pltpu.VMEM((_CHUNK, 2 * _DIM), jnp.float32),
            pltpu.SemaphoreType.DMA,
        ],
    )
    def _sc_gather(table_hbm, idx_hbm, out_hbm, idx_v, rows_v, sem):
        wid = lax.axis_index("s") * 2 + lax.axis_index("c")
        base = wid * _RPW
        for chunk in range(_RPW // _CHUNK):
            off = base + chunk * _CHUNK
            pltpu.sync_copy(idx_hbm.at[pl.ds(off, _CHUNK)], idx_v)
            pltpu.async_copy(table_hbm.at[idx_v], rows_v, sem).wait()
            pltpu.sync_copy(rows_v, out_hbm.at[pl.ds(off, _CHUNK)])

    return _sc_gather


def _rn(x):
    return jnp.sqrt(jnp.sum(x * x, axis=1))


def _math_body(g_ref, out_ref):
    g = g_ref[...]
    m = _MARGIN
    m1 = _MARGIN1

    def halves(k):
        row = g[k]
        return row[:, :_DIM], jnp.abs(row[:, _DIM:])

    c1, cr = halves(0)
    d1, dr = halves(1)
    euc = jnp.abs(c1 - d1)
    nf1 = (_rn(jnp.maximum(euc + cr - dr + m1, 0.0))
           + _rn(jnp.maximum(m - cr, 0.0)) + _rn(jnp.maximum(m - dr, 0.0)))

    c1, c2 = halves(2)
    d1, d2 = halves(3)
    e1, er = halves(4)
    start_all = jnp.maximum(c1 - c2, d1 - d2)
    end_all = jnp.minimum(c1 + c2, d1 + d2)
    new_r = (end_all - start_all) * 0.5
    cen1 = (start_all + end_all) * 0.5
    euc = jnp.abs(cen1 - e1)
    nf2 = (_rn(jnp.maximum(euc + new_r - er + m1, 0.0))
           + _rn(jnp.maximum(start_all - end_all, 0.0)))

    c1, cr = halves(5)
    d1, dr = halves(6)
    r1 = g[7][:, :_DIM]
    delta = jnp.abs(g[7][:, _DIM])
    euc = jnp.abs(c1 + r1 - d1)
    nf3 = (_rn(jnp.maximum(euc + cr - dr + m1 - delta[:, None], 0.0))
           + _rn(jnp.maximum(m - cr, 0.0)) + _rn(jnp.maximum(m - dr, 0.0))
           + delta)

    r1 = g[8][:, :_DIM]
    delta = jnp.abs(g[8][:, _DIM])
    c1, cr = halves(9)
    d1, dr = halves(10)
    euc = jnp.abs(c1 - r1 - d1)
    nf4 = (_rn(jnp.maximum(euc - cr - dr + m1 + delta[:, None], 0.0))
           + _rn(jnp.maximum(m - cr, 0.0)) + _rn(jnp.maximum(m - dr, 0.0))
           + delta)

    c1, cr = halves(11)
    d1, dr = halves(12)
    euc = jnp.abs(c1 - d1)
    dis = (_rn(jnp.maximum(-euc + cr + dr + m1, 0.0))
           + _rn(jnp.maximum(m - cr, 0.0)) + _rn(jnp.maximum(m - dr, 0.0)))

    c1, cr = halves(13)
    d1, dr = halves(14)
    euc = jnp.abs(c1 - d1)
    neg = (_rn(jnp.maximum(euc - cr - dr - m1, 0.0))
           + _rn(jnp.maximum(m - cr, 0.0)) + _rn(jnp.maximum(m - dr, 0.0)))

    d1, dr2 = halves(15)
    dr2 = dr2 * 0.5
    topl = _rn(jnp.maximum(_INF - dr2, 0.0)) + _rn(jnp.maximum(_INF + d1, 0.0))

    total = (jnp.mean(nf1) + jnp.mean(nf2) + jnp.mean(nf3) + jnp.mean(nf4)
             + jnp.mean(dis) + jnp.mean(neg) + jnp.mean(topl))
    out_ref[0, 0] = total


def _math_call(g):
    return pl.pallas_call(
        _math_body,
        out_shape=jax.ShapeDtypeStruct((1, 1), jnp.float32),
        out_specs=pl.BlockSpec(memory_space=pltpu.SMEM),
    )(g)


def kernel(class_emb, rel_emb, nf1, nf2, nf3, nf4, disjoint, neg, top):
    re_pad = jnp.pad(rel_emb, ((0, 0), (0, 2 * _DIM - (_DIM + 1))))
    table = jnp.concatenate([class_emb, re_pad], axis=0)
    idx = jnp.concatenate([
        nf1[:_B, 0], nf1[:_B, 1],
        nf2[:_B, 0], nf2[:_B, 1], nf2[:_B, 2],
        nf3[:_B, 0], nf3[:_B, 2], nf3[:_B, 1] + 1000,
        nf4[:_B, 0] + 1000, nf4[:_B, 1], nf4[:_B, 2],
        disjoint[:_B, 0], disjoint[:_B, 1],
        neg[:_B, 0], neg[:_B, 1],
        top[:_B],
    ])
    rows = _get_sc_gather()(table, idx)
    out = _math_call(rows.reshape(_NSLOT, _B, 2 * _DIM))
    return out[0, 0]
